# Initial kernel scaffold; baseline (speedup 1.0000x reference)
import jax
import jax.numpy as jnp
from jax.experimental import pallas as pl


def kernel(fused_proto, base_proto, logits):
    raise NotImplementedError("write your pallas kernel here")


# speedup vs baseline: 50.1054x; 50.1054x over previous
import functools

import jax
import jax.numpy as jnp
from jax.experimental import pallas as pl
from jax.experimental.pallas import tpu as pltpu

_RATIOS = (0.1, 0.25, 0.5)
_ROWS_PER_BLOCK = 256


def _gate_kernel(w_ref, f_ref, b_ref, o_ref, *, ks):
    f = f_ref[...]
    b = b_ref[...]
    delta = jnp.abs(f - b)
    bits = jax.lax.bitcast_convert_type(delta, jnp.int32)
    rows = f.shape[0]

    weight = jnp.zeros(f.shape, jnp.float32)
    for i, k in enumerate(ks):
        def body(j, prefix, k=k):
            cand = prefix | jax.lax.shift_left(jnp.int32(1), 30 - j)
            cnt = jnp.sum((bits >= cand).astype(jnp.int32), axis=1,
                          keepdims=True)
            return jnp.where(cnt >= k, cand, prefix)

        thresh = jax.lax.fori_loop(
            0, 31, body, jnp.zeros((rows, 1), jnp.int32))
        weight = weight + jnp.where(bits >= thresh, w_ref[i], 0.0)
    o_ref[...] = f * weight


def kernel(fused_proto, base_proto, logits):
    q, n, d = fused_proto.shape
    rows = q * n
    ks = tuple(max(1, int(r * d)) for r in _RATIOS)
    w = jax.nn.softmax(logits, axis=0)

    f2 = fused_proto.reshape(rows, d)
    b2 = base_proto.reshape(rows, d)
    rb = _ROWS_PER_BLOCK
    grid = (rows // rb,)
    out = pl.pallas_call(
        functools.partial(_gate_kernel, ks=ks),
        grid=grid,
        in_specs=[
            pl.BlockSpec(memory_space=pltpu.SMEM),
            pl.BlockSpec((rb, d), lambda i: (i, 0)),
            pl.BlockSpec((rb, d), lambda i: (i, 0)),
        ],
        out_specs=pl.BlockSpec((rb, d), lambda i: (i, 0)),
        out_shape=jax.ShapeDtypeStruct((rows, d), jnp.float32),
    )(w, f2, b2)
    return out.reshape(q, n, d)


# speedup vs baseline: 61.6104x; 1.2296x over previous
import functools

import jax
import jax.numpy as jnp
from jax.experimental import pallas as pl
from jax.experimental.pallas import tpu as pltpu

_RATIOS = (0.1, 0.25, 0.5)
_ROWS_PER_BLOCK = 256


def _gate_kernel(w_ref, f_ref, b_ref, o_ref, *, ks):
    f = f_ref[...]
    b = b_ref[...]
    delta = jnp.abs(f - b)
    bits = jax.lax.bitcast_convert_type(delta, jnp.int32)
    rows = f.shape[0]

    def body(j, prefixes, ks=ks):
        bit = jax.lax.shift_left(jnp.int32(1), 30 - j)
        new = []
        for k, prefix in zip(ks, prefixes):
            cand = prefix | bit
            cnt = jnp.sum((bits >= cand).astype(jnp.int32), axis=1,
                          keepdims=True)
            new.append(jnp.where(cnt >= k, cand, prefix))
        return tuple(new)

    zero = jnp.zeros((rows, 1), jnp.int32)
    threshs = jax.lax.fori_loop(0, 31, body, (zero, zero, zero))
    weight = jnp.zeros(f.shape, jnp.float32)
    for i, thresh in enumerate(threshs):
        weight = weight + jnp.where(bits >= thresh, w_ref[i], 0.0)
    o_ref[...] = f * weight


def kernel(fused_proto, base_proto, logits):
    q, n, d = fused_proto.shape
    rows = q * n
    ks = tuple(max(1, int(r * d)) for r in _RATIOS)
    w = jax.nn.softmax(logits, axis=0)

    f2 = fused_proto.reshape(rows, d)
    b2 = base_proto.reshape(rows, d)
    rb = _ROWS_PER_BLOCK
    grid = (rows // rb,)
    out = pl.pallas_call(
        functools.partial(_gate_kernel, ks=ks),
        grid=grid,
        in_specs=[
            pl.BlockSpec(memory_space=pltpu.SMEM),
            pl.BlockSpec((rb, d), lambda i: (i, 0)),
            pl.BlockSpec((rb, d), lambda i: (i, 0)),
        ],
        out_specs=pl.BlockSpec((rb, d), lambda i: (i, 0)),
        out_shape=jax.ShapeDtypeStruct((rows, d), jnp.float32),
    )(w, f2, b2)
    return out.reshape(q, n, d)
